# baseline (speedup 1.0000x reference)
import functools

import jax
import jax.numpy as jnp
from jax import lax
from jax.experimental import pallas as pl
from jax.experimental.pallas import tpu as pltpu
from jax.experimental.pallas import tpu_sc as plsc

N = 10000
NPAD = 10112
E = 320000
DIN = 128
DOUT = 128
CHUNK = 128
EDGES_PER_CORE = E // 2
CPC = 1280
PAD_TAIL = CPC * CHUNK - EDGES_PER_CORE
NS = 16
CPT = CPC // NS
ROWS_PER_TILE = NPAD // NS
ZB = (128, 128, 128, 128, 120)


def _mm_body(x_ref, w_ref, o_ref):
    o_ref[...] = jnp.dot(x_ref[...], w_ref[...], preferred_element_type=jnp.float32)


def _matmul(x, W):
    bm = 1000
    return pl.pallas_call(
        _mm_body,
        grid=(N // bm,),
        in_specs=[
            pl.BlockSpec((bm, DIN), lambda i: (i, 0)),
            pl.BlockSpec((DIN, DOUT), lambda i: (0, 0)),
        ],
        out_specs=pl.BlockSpec((bm, DOUT), lambda i: (i, 0)),
        out_shape=jax.ShapeDtypeStruct((N, DOUT), jnp.float32),
    )(x, W)


def _combine_body(p_ref, o_ref):
    o_ref[...] = jnp.maximum(p_ref[0] + p_ref[1], 0.0)


def _combine_relu(partials):
    bm = 1000
    return pl.pallas_call(
        _combine_body,
        grid=(N // bm,),
        in_specs=[pl.BlockSpec((2, bm, DOUT), lambda i: (0, i, 0))],
        out_specs=pl.BlockSpec((bm, DOUT), lambda i: (i, 0)),
        out_shape=jax.ShapeDtypeStruct((N, DOUT), jnp.float32),
    )(partials)


@functools.partial(
    pl.kernel,
    out_type=jax.ShapeDtypeStruct((2, NPAD, DOUT), jnp.float32),
    mesh=plsc.VectorSubcoreMesh(core_axis_name="c", subcore_axis_name="s"),
    scratch_types=[
        pltpu.VMEM((CHUNK,), jnp.int32),
        pltpu.VMEM((CHUNK,), jnp.int32),
        pltpu.VMEM((CHUNK,), jnp.int32),
        pltpu.VMEM((CHUNK,), jnp.int32),
        pltpu.VMEM((CHUNK,), jnp.int32),
        pltpu.VMEM((CHUNK,), jnp.float32),
        pltpu.VMEM((CHUNK,), jnp.float32),
        pltpu.VMEM((CHUNK,), jnp.float32),
        pltpu.VMEM((CHUNK, DOUT), jnp.float32),
        pltpu.VMEM((CHUNK, DOUT), jnp.float32),
        pltpu.VMEM((CHUNK, DOUT), jnp.float32),
        pltpu.VMEM_SHARED((NPAD, DOUT), jnp.float32),
        pltpu.SemaphoreType.DMA,
        pltpu.SemaphoreType.DMA,
        pltpu.SemaphoreType.DMA,
        pltpu.SemaphoreType.DMA,
        pltpu.SemaphoreType.DMA,
        pltpu.SemaphoreType.DMA,
        pltpu.SemaphoreType.DMA,
        pltpu.SemaphoreType.DMA,
        pltpu.SemaphoreType.DMA,
    ],
)
def _sc_aggregate(pre_hbm, src_hbm, dst_hbm, ew_hbm, out_hbm,
                  src_0, src_1, dst_0, dst_1, dst_2, ew_0, ew_1, ew_2,
                  rows_0, rows_1, rows_2, acc,
                  sem_src0, sem_src1, sem_de0, sem_de1, sem_de2,
                  sem_g, sem_s0, sem_s1, sem_s2):
    c = lax.axis_index("c")
    s = lax.axis_index("s")
    row0 = s * ROWS_PER_TILE
    SRC = (src_0, src_1)
    SSEM = (sem_src0, sem_src1)
    DST = (dst_0, dst_1, dst_2)
    EW = (ew_0, ew_1, ew_2)
    DSEM = (sem_de0, sem_de1, sem_de2)
    ROWS = (rows_0, rows_1, rows_2)
    CSEM = (sem_s0, sem_s1, sem_s2)

    def _e0(i):
        return (c * CPC + s + i * NS) * CHUNK

    def _src_cp(i, k):
        return pltpu.make_async_copy(
            src_hbm.at[pl.ds(_e0(i), CHUNK)], SRC[k], SSEM[k])

    def _de_cps(i, k):
        return (
            pltpu.make_async_copy(
                dst_hbm.at[pl.ds(_e0(i), CHUNK)], DST[k], DSEM[k]),
            pltpu.make_async_copy(
                ew_hbm.at[pl.ds(_e0(i), CHUNK)], EW[k], DSEM[k]),
        )

    def _gth(k2, k3):
        return pltpu.make_async_copy(pre_hbm.at[SRC[k2]], ROWS[k3], sem_g)

    def _sct(k):
        return pltpu.make_async_copy(ROWS[k], acc.at[DST[k]], CSEM[k])

    def _scale(k):
        wv, rv = EW[k], ROWS[k]

        def body(eg, carry2):
            w16 = wv[pl.ds(eg * 16, 16)]
            for k in range(16):
                e = eg * 16 + k
                wk = w16[k]
                for j in range(DOUT // 16):
                    sl = pl.ds(j * 16, 16)
                    rv[e, sl] = rv[e, sl] * wk
            return carry2

        lax.fori_loop(0, CHUNK // 16, body, 0)

    def _zero_row(r, carry):
        for j in range(DOUT // 16):
            rows_0[r, pl.ds(j * 16, 16)] = jnp.zeros((16,), jnp.float32)
        return carry

    lax.fori_loop(0, 128, _zero_row, 0)
    off = 0
    for zb in ZB:
        pltpu.sync_copy(rows_0.at[pl.ds(0, zb)],
                        acc.at[pl.ds(row0 + off, zb)])
        off += zb
    plsc.subcore_barrier()

    def _slot(t, w):
        i = 2 + 6 * t + w
        a3, a2 = (2 + w) % 3, w % 2
        b3, b2 = (3 + w) % 3, (w + 1) % 2
        _gth(a2, a3).wait()
        _sct(w % 3).wait()

        @pl.when(i + 1 < CPT)
        def _():
            for cp in _de_cps(i + 1, b3):
                cp.start()
            _src_cp(i + 1, b2).wait()
            _gth(b2, b3).start()

        @pl.when(i + 2 < CPT)
        def _():
            _src_cp(i + 2, a2).start()

        for cp in _de_cps(i, a3):
            cp.wait()
        _scale(a3)
        pltpu.async_copy(ROWS[a3], acc.at[DST[a3]], CSEM[a3], add=True)

    _src_cp(0, 0).start()
    _src_cp(1, 1).start()
    for j in range(3):
        for cp in _de_cps(j, j):
            cp.start()
    _src_cp(0, 0).wait()
    _gth(0, 0).start()
    _gth(0, 0).wait()
    _src_cp(1, 1).wait()
    _gth(1, 1).start()
    _src_cp(2, 0).start()
    for cp in _de_cps(0, 0):
        cp.wait()
    _scale(0)
    pltpu.async_copy(ROWS[0], acc.at[DST[0]], CSEM[0], add=True)

    _gth(1, 1).wait()
    _src_cp(2, 0).wait()
    _gth(0, 2).start()
    _src_cp(3, 1).start()
    for cp in _de_cps(1, 1):
        cp.wait()
    _scale(1)
    pltpu.async_copy(ROWS[1], acc.at[DST[1]], CSEM[1], add=True)

    def _six(t, carry):
        for w in range(6):
            _slot(t, w)
        return carry

    lax.fori_loop(0, (CPT - 2) // 6, _six, 0)
    _sct(0).wait()
    _sct(1).wait()
    plsc.subcore_barrier()

    pltpu.sync_copy(acc.at[pl.ds(row0, ROWS_PER_TILE)],
                    out_hbm.at[c, pl.ds(row0, ROWS_PER_TILE)])


def _pad_split(a):
    z = jnp.zeros((PAD_TAIL,), a.dtype)
    return jnp.concatenate([a[:EDGES_PER_CORE], z, a[EDGES_PER_CORE:], z])


def kernel(x, edge_index, edge_weight, W):
    pre = _matmul(x, W)
    partials = _sc_aggregate(
        pre,
        _pad_split(edge_index[0]),
        _pad_split(edge_index[1]),
        _pad_split(edge_weight),
    )
    return _combine_relu(partials)


# speedup vs baseline: 2.3545x; 2.3545x over previous
import functools

import jax
import jax.numpy as jnp
from jax import lax
from jax.experimental import pallas as pl
from jax.experimental.pallas import tpu as pltpu
from jax.experimental.pallas import tpu_sc as plsc

N = 10000
NPAD = 10240
E = 320000
DIN = 128
DOUT = 128
CHUNK = 128
EDGES_PER_CORE = E // 2
NUM_CHUNKS = EDGES_PER_CORE // CHUNK
NS = 16
ROWS_PER_TILE = NPAD // NS
RB = 128
CHUNKS_PER_TILE = -(-NUM_CHUNKS // NS)


def _mm_body(x_ref, w_ref, o_ref):
    o_ref[...] = jnp.dot(x_ref[...], w_ref[...], preferred_element_type=jnp.float32)


def _matmul(x, W):
    bm = 2000
    return pl.pallas_call(
        _mm_body,
        grid=(N // bm,),
        in_specs=[
            pl.BlockSpec((bm, DIN), lambda i: (i, 0)),
            pl.BlockSpec((DIN, DOUT), lambda i: (0, 0)),
        ],
        out_specs=pl.BlockSpec((bm, DOUT), lambda i: (i, 0)),
        out_shape=jax.ShapeDtypeStruct((N, DOUT), jnp.float32),
    )(x, W)


def _combine_body(p_ref, o_ref):
    o_ref[...] = jnp.maximum(p_ref[0] + p_ref[1], 0.0)


def _combine_relu(partials):
    bm = 2000
    return pl.pallas_call(
        _combine_body,
        grid=(N // bm,),
        in_specs=[pl.BlockSpec((2, bm, DOUT), lambda i: (0, i, 0))],
        out_specs=pl.BlockSpec((bm, DOUT), lambda i: (i, 0)),
        out_shape=jax.ShapeDtypeStruct((N, DOUT), jnp.float32),
    )(partials)


@functools.partial(
    pl.kernel,
    out_type=jax.ShapeDtypeStruct((2, NPAD, DOUT), jnp.float32),
    mesh=plsc.VectorSubcoreMesh(core_axis_name="c", subcore_axis_name="s"),
    scratch_types=[
        pltpu.VMEM((CHUNK,), jnp.int32),
        pltpu.VMEM((CHUNK,), jnp.int32),
        pltpu.VMEM((CHUNK,), jnp.float32),
        pltpu.VMEM((CHUNK,), jnp.int32),
        pltpu.VMEM((CHUNK,), jnp.int32),
        pltpu.VMEM((CHUNK,), jnp.float32),
        pltpu.VMEM((CHUNK, DOUT), jnp.float32),
        pltpu.VMEM((CHUNK, DOUT), jnp.float32),
        pltpu.VMEM_SHARED((NPAD, DOUT), jnp.float32),
        pltpu.SemaphoreType.DMA,
        pltpu.SemaphoreType.DMA,
        pltpu.SemaphoreType.DMA,
    ],
)
def _sc_aggregate(pre_hbm, src_hbm, dst_hbm, ew_hbm, out_hbm,
                  src_a, dst_a, ew_a, src_b, dst_b, ew_b,
                  rows_a, rows_b, acc, sem_ia, sem_ib, sem_g):
    c = lax.axis_index("c")
    s = lax.axis_index("s")
    row0 = s * ROWS_PER_TILE

    def _idx_copies(i, sv, dv, wv, sem):
        g = s + i * NS
        e0 = c * EDGES_PER_CORE + g * CHUNK
        return (
            pltpu.make_async_copy(src_hbm.at[pl.ds(e0, CHUNK)], sv, sem),
            pltpu.make_async_copy(dst_hbm.at[pl.ds(e0, CHUNK)], dv, sem),
            pltpu.make_async_copy(ew_hbm.at[pl.ds(e0, CHUNK)], wv, sem),
        )

    def _start_idx(i, sv, dv, wv, sem):
        @pl.when(s + i * NS < NUM_CHUNKS)
        def _():
            for cp in _idx_copies(i, sv, dv, wv, sem):
                cp.start()

    def _scale(wv, rv):
        def body(eg, carry2):
            w16 = wv[pl.ds(eg * 16, 16)]
            for k in range(16):
                e = eg * 16 + k
                wk = w16[k]
                for j in range(DOUT // 16):
                    sl = pl.ds(j * 16, 16)
                    rv[e, sl] = rv[e, sl] * wk
            return carry2

        lax.fori_loop(0, CHUNK // 16, body, 0)

    def _zero_row(r, carry):
        for j in range(DOUT // 16):
            rows_a[r, pl.ds(j * 16, 16)] = jnp.zeros((16,), jnp.float32)
        return carry

    lax.fori_loop(0, RB, _zero_row, 0)
    for b in range(ROWS_PER_TILE // RB):
        pltpu.sync_copy(rows_a.at[pl.ds(0, RB)],
                        acc.at[pl.ds(row0 + b * RB, RB)])
    plsc.subcore_barrier()

    _start_idx(0, src_a, dst_a, ew_a, sem_ia)
    _start_idx(1, src_b, dst_b, ew_b, sem_ib)

    @pl.when(s < NUM_CHUNKS)
    def _():
        for cp in _idx_copies(0, src_a, dst_a, ew_a, sem_ia):
            cp.wait()
        pltpu.make_async_copy(pre_hbm.at[src_a], rows_a, sem_g).start()

    def _slot(i, sv, dv, wv, sem, rv, nsv, ndv, nwv, nsem, nrv):
        @pl.when(s + i * NS < NUM_CHUNKS)
        def _():
            pltpu.make_async_copy(pre_hbm.at[sv], rv, sem_g).wait()

            @pl.when(s + (i + 1) * NS < NUM_CHUNKS)
            def _():
                for cp in _idx_copies(i + 1, nsv, ndv, nwv, nsem):
                    cp.wait()
                pltpu.make_async_copy(pre_hbm.at[nsv], nrv, sem_g).start()

            _scale(wv, rv)
            pltpu.sync_copy(rv, acc.at[dv], add=True)
            _start_idx(i + 2, sv, dv, wv, sem)

    def _pair(t, carry):
        _slot(2 * t, src_a, dst_a, ew_a, sem_ia, rows_a,
              src_b, dst_b, ew_b, sem_ib, rows_b)
        _slot(2 * t + 1, src_b, dst_b, ew_b, sem_ib, rows_b,
              src_a, dst_a, ew_a, sem_ia, rows_a)
        return carry

    lax.fori_loop(0, (CHUNKS_PER_TILE + 1) // 2, _pair, 0)
    plsc.subcore_barrier()

    pltpu.sync_copy(acc.at[pl.ds(row0, ROWS_PER_TILE)],
                    out_hbm.at[c, pl.ds(row0, ROWS_PER_TILE)])


def kernel(x, edge_index, edge_weight, W):
    pre = _matmul(x, W)
    partials = _sc_aggregate(pre, edge_index[0], edge_index[1], edge_weight)
    return _combine_relu(partials)


# speedup vs baseline: 2.3869x; 1.0138x over previous
import functools

import jax
import jax.numpy as jnp
from jax import lax
from jax.experimental import pallas as pl
from jax.experimental.pallas import tpu as pltpu
from jax.experimental.pallas import tpu_sc as plsc

N = 10000
NPAD = 10240
E = 320000
DIN = 128
DOUT = 128
CHUNK = 128
EDGES_PER_CORE = E // 2
NUM_CHUNKS = EDGES_PER_CORE // CHUNK
NS = 16
ROWS_PER_TILE = NPAD // NS
RB = 128
CHUNKS_PER_TILE = -(-NUM_CHUNKS // NS)


def _mm_body(x_ref, w_ref, o_ref):
    o_ref[...] = jnp.dot(x_ref[...], w_ref[...], preferred_element_type=jnp.float32)


def _matmul(x, W):
    bm = 10000
    return pl.pallas_call(
        _mm_body,
        grid=(N // bm,),
        in_specs=[
            pl.BlockSpec((bm, DIN), lambda i: (i, 0)),
            pl.BlockSpec((DIN, DOUT), lambda i: (0, 0)),
        ],
        out_specs=pl.BlockSpec((bm, DOUT), lambda i: (i, 0)),
        out_shape=jax.ShapeDtypeStruct((N, DOUT), jnp.float32),
    )(x, W)


def _combine_body(p_ref, o_ref):
    o_ref[...] = jnp.maximum(p_ref[0] + p_ref[1], 0.0)


def _combine_relu(partials):
    bm = 10000
    return pl.pallas_call(
        _combine_body,
        grid=(N // bm,),
        in_specs=[pl.BlockSpec((2, bm, DOUT), lambda i: (0, i, 0))],
        out_specs=pl.BlockSpec((bm, DOUT), lambda i: (i, 0)),
        out_shape=jax.ShapeDtypeStruct((N, DOUT), jnp.float32),
    )(partials)


@functools.partial(
    pl.kernel,
    out_type=jax.ShapeDtypeStruct((2, NPAD, DOUT), jnp.float32),
    mesh=plsc.VectorSubcoreMesh(core_axis_name="c", subcore_axis_name="s"),
    scratch_types=[
        pltpu.VMEM((CHUNK,), jnp.int32),
        pltpu.VMEM((CHUNK,), jnp.int32),
        pltpu.VMEM((CHUNK,), jnp.float32),
        pltpu.VMEM((CHUNK,), jnp.int32),
        pltpu.VMEM((CHUNK,), jnp.int32),
        pltpu.VMEM((CHUNK,), jnp.float32),
        pltpu.VMEM((CHUNK, DOUT), jnp.float32),
        pltpu.VMEM((CHUNK, DOUT), jnp.float32),
        pltpu.VMEM_SHARED((NPAD, DOUT), jnp.float32),
        pltpu.SemaphoreType.DMA,
        pltpu.SemaphoreType.DMA,
        pltpu.SemaphoreType.DMA,
    ],
)
def _sc_aggregate(pre_hbm, src_hbm, dst_hbm, ew_hbm, out_hbm,
                  src_a, dst_a, ew_a, src_b, dst_b, ew_b,
                  rows_a, rows_b, acc, sem_ia, sem_ib, sem_g):
    c = lax.axis_index("c")
    s = lax.axis_index("s")
    row0 = s * ROWS_PER_TILE

    def _idx_copies(i, sv, dv, wv, sem):
        g = s + i * NS
        e0 = c * EDGES_PER_CORE + g * CHUNK
        return (
            pltpu.make_async_copy(src_hbm.at[pl.ds(e0, CHUNK)], sv, sem),
            pltpu.make_async_copy(dst_hbm.at[pl.ds(e0, CHUNK)], dv, sem),
            pltpu.make_async_copy(ew_hbm.at[pl.ds(e0, CHUNK)], wv, sem),
        )

    def _start_idx(i, sv, dv, wv, sem):
        @pl.when(s + i * NS < NUM_CHUNKS)
        def _():
            for cp in _idx_copies(i, sv, dv, wv, sem):
                cp.start()

    def _scale(wv, rv):
        def body(eg, carry2):
            w16 = wv[pl.ds(eg * 16, 16)]
            for k in range(16):
                e = eg * 16 + k
                wk = w16[k]
                for j in range(DOUT // 16):
                    sl = pl.ds(j * 16, 16)
                    rv[e, sl] = rv[e, sl] * wk
            return carry2

        lax.fori_loop(0, CHUNK // 16, body, 0)

    def _zero_row(r, carry):
        for j in range(DOUT // 16):
            rows_a[r, pl.ds(j * 16, 16)] = jnp.zeros((16,), jnp.float32)
        return carry

    lax.fori_loop(0, RB, _zero_row, 0)
    for b in range(ROWS_PER_TILE // RB):
        pltpu.sync_copy(rows_a.at[pl.ds(0, RB)],
                        acc.at[pl.ds(row0 + b * RB, RB)])
    plsc.subcore_barrier()

    _start_idx(0, src_a, dst_a, ew_a, sem_ia)
    _start_idx(1, src_b, dst_b, ew_b, sem_ib)

    @pl.when(s < NUM_CHUNKS)
    def _():
        for cp in _idx_copies(0, src_a, dst_a, ew_a, sem_ia):
            cp.wait()
        pltpu.make_async_copy(pre_hbm.at[src_a], rows_a, sem_g).start()

    def _slot(i, sv, dv, wv, sem, rv, nsv, ndv, nwv, nsem, nrv):
        @pl.when(s + i * NS < NUM_CHUNKS)
        def _():
            pltpu.make_async_copy(pre_hbm.at[sv], rv, sem_g).wait()

            @pl.when(s + (i + 1) * NS < NUM_CHUNKS)
            def _():
                for cp in _idx_copies(i + 1, nsv, ndv, nwv, nsem):
                    cp.wait()
                pltpu.make_async_copy(pre_hbm.at[nsv], nrv, sem_g).start()

            _scale(wv, rv)
            pltpu.sync_copy(rv, acc.at[dv], add=True)
            _start_idx(i + 2, sv, dv, wv, sem)

    def _pair(t, carry):
        _slot(2 * t, src_a, dst_a, ew_a, sem_ia, rows_a,
              src_b, dst_b, ew_b, sem_ib, rows_b)
        _slot(2 * t + 1, src_b, dst_b, ew_b, sem_ib, rows_b,
              src_a, dst_a, ew_a, sem_ia, rows_a)
        return carry

    lax.fori_loop(0, (CHUNKS_PER_TILE + 1) // 2, _pair, 0)
    plsc.subcore_barrier()

    pltpu.sync_copy(acc.at[pl.ds(row0, ROWS_PER_TILE)],
                    out_hbm.at[c, pl.ds(row0, ROWS_PER_TILE)])


def kernel(x, edge_index, edge_weight, W):
    pre = _matmul(x, W)
    partials = _sc_aggregate(pre, edge_index[0], edge_index[1], edge_weight)
    return _combine_relu(partials)
